# baseline (speedup 1.0000x reference)
import functools

import jax
import jax.numpy as jnp
from jax import lax
from jax.experimental import pallas as pl
from jax.experimental.pallas import tpu as pltpu
from jax.experimental.pallas import tpu_sc as plsc

NC = 2
NS = 16
NW = NC * NS


def _sc_body(CB, NCH, UC, K, KP, D, u_hbm, v_hbm, idx_hbm,
             su_hbm, sv_hbm, sn_hbm, idx_v, ru, rv, rn, ou, ov, on, sem):
    L = UC // CB
    stride = 2 * UC + CB * KP
    bpw = CB * NCH
    wid = lax.axis_index("s") * NC + lax.axis_index("c")
    pltpu.sync_copy(idx_hbm.at[pl.ds(wid * (NCH * stride), NCH * stride)], idx_v)

    def chunk_body(c, carry):
        off = c * stride
        copies = [
            pltpu.async_copy(u_hbm.at[idx_v.at[pl.ds(off, UC)]], ru, sem),
            pltpu.async_copy(v_hbm.at[idx_v.at[pl.ds(off + UC, UC)]], rv, sem),
        ]
        for j in range(CB):
            copies.append(pltpu.async_copy(
                v_hbm.at[idx_v.at[pl.ds(off + 2 * UC + j * KP, KP)]],
                rn.at[pl.ds(j * KP, KP)], sem))
        for cp in copies:
            cp.wait()

        def b_body(bl, carry2):
            row = c * CB + bl
            for dc in range(D // 16):
                ds_ = pl.ds(dc * 16, 16)
                au = ru[bl * L, ds_]
                av = rv[bl * L, ds_]
                for l in range(1, L):
                    au = au + ru[bl * L + l, ds_]
                    av = av + rv[bl * L + l, ds_]
                ou[row, ds_] = au
                ov[row, ds_] = av
                an = rn[bl * KP, ds_]
                for k in range(1, K):
                    an = an + rn[bl * KP + k, ds_]
                on[row, ds_] = an
            return carry2
        lax.fori_loop(0, CB, b_body, None)
        return carry
    lax.fori_loop(0, NCH, chunk_body, None)

    base = wid * bpw
    pltpu.sync_copy(ou, su_hbm.at[pl.ds(base, bpw)])
    pltpu.sync_copy(ov, sv_hbm.at[pl.ds(base, bpw)])
    pltpu.sync_copy(on, sn_hbm.at[pl.ds(base, bpw)])


def _tc_body(scale, su_ref, sv_ref, sn_ref, o_ref):
    su = su_ref[...]
    sv = sv_ref[...]
    sn = sn_ref[...]
    s = jnp.sum(su * sv, axis=1) * scale
    ns = jnp.sum(su * sn, axis=1) * scale

    def logsig(x):
        return jnp.minimum(x, 0.0) - jnp.log1p(jnp.exp(-jnp.abs(x)))

    loss = logsig(s) + logsig(-ns)
    o_ref[...] = jnp.reshape(-jnp.sum(loss), (1, 1))


def kernel(u_emb, v_emb, u_pos, v_pos, v_neg, batch_size):
    B, L = u_pos.shape
    NNEG = v_neg.shape[2]
    D = u_emb.shape[1]
    CB = 4
    BPW = B // NW
    NCH = BPW // CB
    UC = CB * L
    K = L * NNEG
    KP = (K + 7) // 8 * 8
    stride = 2 * UC + CB * KP

    u_i = u_pos.astype(jnp.int32).reshape(NW, NCH, UC)
    v_i = v_pos.astype(jnp.int32).reshape(NW, NCH, UC)
    n_i = v_neg.astype(jnp.int32).reshape(B, K)
    n_i = jnp.pad(n_i, ((0, 0), (0, KP - K))).reshape(NW, NCH, CB * KP)
    idx = jnp.concatenate([u_i, v_i, n_i], axis=-1).reshape(NW * NCH * stride)

    sc = functools.partial(
        pl.kernel,
        mesh=plsc.VectorSubcoreMesh(core_axis_name="c", subcore_axis_name="s"),
        out_type=[jax.ShapeDtypeStruct((B, D), jnp.float32)] * 3,
        scratch_types=[
            pltpu.VMEM((NCH * stride,), jnp.int32),
            pltpu.VMEM((UC, D), jnp.float32),
            pltpu.VMEM((UC, D), jnp.float32),
            pltpu.VMEM((CB * KP, D), jnp.float32),
            pltpu.VMEM((BPW, D), jnp.float32),
            pltpu.VMEM((BPW, D), jnp.float32),
            pltpu.VMEM((BPW, D), jnp.float32),
            pltpu.SemaphoreType.DMA,
        ],
        compiler_params=pltpu.CompilerParams(use_tc_tiling_on_sc=False),
    )(functools.partial(_sc_body, CB, NCH, UC, K, KP, D))
    su, sv, sn = sc(u_emb, v_emb, idx)

    out = pl.pallas_call(
        functools.partial(_tc_body, 1.0 / float(L * L)),
        out_shape=jax.ShapeDtypeStruct((1, 1), jnp.float32),
    )(su, sv, sn)
    return out[0, 0] / jnp.asarray(batch_size, jnp.float32)


# speedup vs baseline: 1.2041x; 1.2041x over previous
import functools

import jax
import jax.numpy as jnp
from jax import lax
from jax.experimental import pallas as pl
from jax.experimental.pallas import tpu as pltpu
from jax.experimental.pallas import tpu_sc as plsc

NC = 2
NS = 16
NW = NC * NS


def _gather_pool_pass(tbl, idx_ref, idx_base, G, CBs, nch, rb0, rb1,
                      out_ref, sem0, sem1, out_base):
    nrows = CBs * G

    def fire(c, rb, sem):
        off = idx_base + c * nrows
        pltpu.async_copy(tbl.at[idx_ref.at[pl.ds(off, nrows)]],
                         rb.at[pl.ds(0, nrows)], sem)

    def drain(rb, sem):
        pltpu.make_async_copy(tbl.at[pl.ds(0, nrows)],
                              rb.at[pl.ds(0, nrows)], sem).wait()

    def compute(c, rb):
        def b_body(bl, _):
            row = out_base + c * CBs + bl
            base = bl * G
            accs = [[rb[base + p, pl.ds(dc * 16, 16)] for p in range(2)]
                    for dc in range(4)]
            for k in range(2, G):
                p = k % 2
                for dc in range(4):
                    accs[dc][p] = accs[dc][p] + rb[base + k, pl.ds(dc * 16, 16)]
            for dc in range(4):
                out_ref[row, pl.ds(dc * 16, 16)] = accs[dc][0] + accs[dc][1]
            return _
        lax.fori_loop(0, CBs, b_body, None)

    fire(0, rb0, sem0)

    def body2(j, _):
        c0 = 2 * j
        drain(rb0, sem0)
        fire(c0 + 1, rb1, sem1)
        compute(c0, rb0)
        drain(rb1, sem1)
        fire(jnp.minimum(c0 + 2, nch - 1), rb0, sem0)
        compute(c0 + 1, rb1)
        return _
    lax.fori_loop(0, nch // 2, body2, None)
    drain(rb0, sem0)


def _sc_body(B, L, K, D, u_hbm, v_hbm, up_hbm, vp_hbm, vn_hbm,
             su_hbm, sv_hbm, sn_hbm, iu, iv, inb, rb0, rb1,
             ou, ov, on, sem0, sem1, semi):
    bpw = B // NW
    wid = lax.axis_index("s") * NC + lax.axis_index("c")

    pltpu.async_copy(up_hbm.at[pl.ds(wid * (bpw * L), bpw * L)], iu, semi)
    pltpu.async_copy(vp_hbm.at[pl.ds(wid * (bpw * L), bpw * L)], iv, semi)
    pltpu.async_copy(vn_hbm.at[pl.ds(wid * (bpw * K), bpw * K)], inb, semi)
    pltpu.make_async_copy(up_hbm.at[pl.ds(0, bpw * L)], iu, semi).wait()
    pltpu.make_async_copy(vp_hbm.at[pl.ds(0, bpw * L)], iv, semi).wait()
    pltpu.make_async_copy(vn_hbm.at[pl.ds(0, bpw * K)], inb, semi).wait()

    CB_UV = 16
    CB_N = 4
    _gather_pool_pass(u_hbm, iu, 0, L, CB_UV, bpw // CB_UV, rb0, rb1,
                      ou, sem0, sem1, 0)
    _gather_pool_pass(v_hbm, iv, 0, L, CB_UV, bpw // CB_UV, rb0, rb1,
                      ov, sem0, sem1, 0)
    _gather_pool_pass(v_hbm, inb, 0, K, CB_N, bpw // CB_N, rb0, rb1,
                      on, sem0, sem1, 0)

    base = wid * bpw
    pltpu.sync_copy(ou, su_hbm.at[pl.ds(base, bpw)])
    pltpu.sync_copy(ov, sv_hbm.at[pl.ds(base, bpw)])
    pltpu.sync_copy(on, sn_hbm.at[pl.ds(base, bpw)])


def _tc_body(scale, su_ref, sv_ref, sn_ref, o_ref):
    su = su_ref[...]
    sv = sv_ref[...]
    sn = sn_ref[...]
    s = jnp.sum(su * sv, axis=1) * scale
    ns = jnp.sum(su * sn, axis=1) * scale

    def logsig(x):
        return jnp.minimum(x, 0.0) - jnp.log1p(jnp.exp(-jnp.abs(x)))

    loss = logsig(s) + logsig(-ns)
    o_ref[...] = jnp.reshape(-jnp.sum(loss), (1, 1))


def kernel(u_emb, v_emb, u_pos, v_pos, v_neg, batch_size):
    B, L = u_pos.shape
    NNEG = v_neg.shape[2]
    D = u_emb.shape[1]
    K = L * NNEG
    bpw = B // NW

    up = u_pos.astype(jnp.int32).reshape(-1)
    vp = v_pos.astype(jnp.int32).reshape(-1)
    vn = v_neg.astype(jnp.int32).reshape(-1)

    sc = functools.partial(
        pl.kernel,
        mesh=plsc.VectorSubcoreMesh(core_axis_name="c", subcore_axis_name="s"),
        out_type=[jax.ShapeDtypeStruct((B, D), jnp.float32)] * 3,
        scratch_types=[
            pltpu.VMEM((bpw * L,), jnp.int32),
            pltpu.VMEM((bpw * L,), jnp.int32),
            pltpu.VMEM((bpw * K,), jnp.int32),
            pltpu.VMEM((400, D), jnp.float32),
            pltpu.VMEM((400, D), jnp.float32),
            pltpu.VMEM((bpw, D), jnp.float32),
            pltpu.VMEM((bpw, D), jnp.float32),
            pltpu.VMEM((bpw, D), jnp.float32),
            pltpu.SemaphoreType.DMA,
            pltpu.SemaphoreType.DMA,
            pltpu.SemaphoreType.DMA,
        ],
        compiler_params=pltpu.CompilerParams(use_tc_tiling_on_sc=False),
    )(functools.partial(_sc_body, B, L, K, D))
    su, sv, sn = sc(u_emb, v_emb, up, vp, vn)

    out = pl.pallas_call(
        functools.partial(_tc_body, 1.0 / float(L * L)),
        out_shape=jax.ShapeDtypeStruct((1, 1), jnp.float32),
    )(su, sv, sn)
    return out[0, 0] / jnp.asarray(batch_size, jnp.float32)
